# baseline (speedup 1.0000x reference)
import functools

import jax
import jax.numpy as jnp
from jax import lax
from jax.experimental import pallas as pl
from jax.experimental.pallas import tpu as pltpu
from jax.experimental.pallas import tpu_sc as plsc

_NUM_CORES = 2
_NUM_SUBCORES = 16
_LANES = 16
_CHUNK = 128


def kernel(token_ids, tok_table, pos_table):
    B, S = token_ids.shape
    V, E = tok_table.shape
    n_chunks = S // _CHUNK
    half = S // 2

    ids3 = token_ids.reshape(B, n_chunks, _CHUNK)

    mesh = plsc.VectorSubcoreMesh(
        core_axis_name="c",
        subcore_axis_name="s",
        num_cores=_NUM_CORES,
        num_subcores=_NUM_SUBCORES,
    )

    @functools.partial(
        pl.kernel,
        out_type=jax.ShapeDtypeStruct((B, S, E), jnp.float32),
        mesh=mesh,
        scratch_types=[
            pltpu.VMEM((n_chunks, _CHUNK), jnp.int32),
            pltpu.VMEM((S, E), jnp.float32),
            pltpu.VMEM((half, E), jnp.float32),
            pltpu.SemaphoreType.DMA,
        ],
        compiler_params=pltpu.CompilerParams(use_tc_tiling_on_sc=False),
    )
    def run(ids_hbm, tok_hbm, pos_hbm, out_hbm, idx_v, buf, pos_v, sem):
        w = lax.axis_index("s") * _NUM_CORES + lax.axis_index("c")

        pltpu.sync_copy(ids_hbm.at[w], idx_v)

        copies = []
        for c in range(n_chunks):
            copies.append(
                pltpu.async_copy(
                    tok_hbm.at[idx_v.at[c]],
                    buf.at[pl.ds(c * _CHUNK, _CHUNK)],
                    sem,
                )
            )
        for cp in copies:
            cp.wait()

        for h in range(2):
            pltpu.sync_copy(pos_hbm.at[pl.ds(h * half, half)], pos_v)

            def body(r, carry, h=h):
                row = h * half + r
                for q in range(E // _LANES):
                    sl = pl.ds(q * _LANES, _LANES)
                    buf[row, sl] = buf[row, sl] + pos_v[r, sl]
                return carry

            lax.fori_loop(0, half, body, 0, unroll=4)

        pltpu.sync_copy(buf, out_hbm.at[w])

    return run(ids3, tok_table, pos_table)
